# baseline (speedup 1.0000x reference)
import jax
import jax.numpy as jnp
from jax import lax
from jax.experimental import pallas as pl
from jax.experimental.pallas import tpu as pltpu
from jax.experimental.pallas import tpu_sc as plsc

N = 10000
F = 128
E = 320000
NC = 2
NS = 16
NW = NC * NS
CH = 128
CPT = 2 * (-(-E // (CH * NW * 2)))
EP = NW * CPT * CH
T = 10240
RPS = T // NS

_HIGH = lax.Precision.HIGHEST

_sc_mesh = plsc.VectorSubcoreMesh(
    core_axis_name="c", subcore_axis_name="s", num_cores=NC, num_subcores=NS
)


ZR = 16


def _zero_slice(zb, acc, s):
    zv = jnp.zeros((16,), jnp.float32)
    for i in range(ZR):
        for j in range(F // 16):
            zb[i, pl.ds(j * 16, 16)] = zv

    for r in range(RPS // ZR):
        pltpu.sync_copy(zb, acc.at[pl.ds(s * RPS + r * ZR, ZR)])


def _sc_scatter_body(tp, srcb, dstb, out, sidx, didx, rows, zb, acc, sem):
    c = lax.axis_index("c")
    s = lax.axis_index("s")
    wid = s * NC + c
    pltpu.sync_copy(srcb.at[wid], sidx)
    pltpu.sync_copy(dstb.at[wid], didx)
    _zero_slice(zb, acc, s)
    plsc.subcore_barrier()

    def step(j, carry):
        pltpu.async_copy(tp.at[sidx.at[j]], rows, sem).wait()
        pltpu.sync_copy(rows, acc.at[didx.at[j]], add=True)
        return carry

    lax.fori_loop(0, CPT, step, 0)
    plsc.subcore_barrier()
    pltpu.sync_copy(acc.at[pl.ds(s * RPS, RPS)], out.at[c, pl.ds(s * RPS, RPS)])


_sc_scatter = pl.kernel(
    _sc_scatter_body,
    out_type=jax.ShapeDtypeStruct((NC, T, F), jnp.float32),
    mesh=_sc_mesh,
    scratch_types=[
        pltpu.VMEM((CPT, CH), jnp.int32),
        pltpu.VMEM((CPT, CH), jnp.int32),
        pltpu.VMEM((CH, F), jnp.float32),
        pltpu.VMEM((ZR, F), jnp.float32),
        pltpu.VMEM_SHARED((T, F), jnp.float32),
        pltpu.SemaphoreType.DMA,
    ],
)


def _sc_deg_body(dstb, out, didx, ones, zb, deg):
    c = lax.axis_index("c")
    s = lax.axis_index("s")
    wid = s * NC + c
    pltpu.sync_copy(dstb.at[wid], didx)
    ov = jnp.full((16,), 1.0, jnp.float32)

    for i in range(CH):
        for j in range(F // 16):
            ones[i, pl.ds(j * 16, 16)] = ov
    _zero_slice(zb, deg, s)
    plsc.subcore_barrier()

    def step(j, carry):
        pltpu.sync_copy(ones, deg.at[didx.at[j]], add=True)
        return carry

    lax.fori_loop(0, CPT, step, 0)
    plsc.subcore_barrier()
    pltpu.sync_copy(deg.at[pl.ds(s * RPS, RPS)], out.at[c, pl.ds(s * RPS, RPS)])


_sc_deg = pl.kernel(
    _sc_deg_body,
    out_type=jax.ShapeDtypeStruct((NC, T, F), jnp.float32),
    mesh=_sc_mesh,
    scratch_types=[
        pltpu.VMEM((CPT, CH), jnp.int32),
        pltpu.VMEM((CH, F), jnp.float32),
        pltpu.VMEM((ZR, F), jnp.float32),
        pltpu.VMEM_SHARED((T, F), jnp.float32),
    ],
)


R = 1000
BI = 200


def _dinv_body(degp_ref, out_ref):
    cnt = degp_ref[0, :, 0:1] + degp_ref[1, :, 0:1] + 1.0
    out_ref[...] = jnp.broadcast_to(lax.rsqrt(cnt), (R, F))


_dinv_call = pl.pallas_call(
    _dinv_body,
    grid=(N // R,),
    in_specs=[pl.BlockSpec((NC, R, F), lambda i: (0, i, 0))],
    out_specs=pl.BlockSpec((R, F), lambda i: (i, 0)),
    out_shape=jax.ShapeDtypeStruct((N, F), jnp.float32),
)


def _prep_body(x_ref, w_ref, dinv_ref, out_ref):
    out_ref[...] = dinv_ref[...] * jnp.dot(
        x_ref[...], w_ref[...], preferred_element_type=jnp.float32,
        precision=_HIGH)


_prep_call = pl.pallas_call(
    _prep_body,
    grid=(N // R,),
    in_specs=[
        pl.BlockSpec((R, F), lambda i: (i, 0)),
        pl.BlockSpec((F, F), lambda i: (0, 0)),
        pl.BlockSpec((R, F), lambda i: (i, 0)),
    ],
    out_specs=pl.BlockSpec((R, F), lambda i: (i, 0)),
    out_shape=jax.ShapeDtypeStruct((N, F), jnp.float32),
)


def _relu_combine(acc_ref, tp_ref, dinv_ref, b_ref):
    dv = dinv_ref[...]
    return dv, jnp.maximum(
        dv * (acc_ref[0] + acc_ref[1] + tp_ref[...]) + b_ref[...], 0.0)


def _comb_body(acc_ref, tp_ref, dinv_ref, b_ref, w_ref, out_ref):
    dv, h = _relu_combine(acc_ref, tp_ref, dinv_ref, b_ref)
    out_ref[...] = dv * jnp.dot(
        h, w_ref[...], preferred_element_type=jnp.float32, precision=_HIGH)


def _comb2_body(acc_ref, tp_ref, dinv_ref, b_ref, w1_ref, w2_ref,
                o1_ref, o2_ref):
    dv, h = _relu_combine(acc_ref, tp_ref, dinv_ref, b_ref)
    o1_ref[...] = dv * jnp.dot(
        h, w1_ref[...], preferred_element_type=jnp.float32, precision=_HIGH)
    o2_ref[...] = dv * jnp.dot(
        h, w2_ref[...], preferred_element_type=jnp.float32, precision=_HIGH)


def _final_body(acc_ref, tp_ref, dinv_ref, b_ref, out_ref):
    _, h = _relu_combine(acc_ref, tp_ref, dinv_ref, b_ref)
    out_ref[...] = h


_acc_spec = pl.BlockSpec((NC, R, F), lambda i: (0, i, 0))
_row_spec = pl.BlockSpec((R, F), lambda i: (i, 0))
_b_spec = pl.BlockSpec((1, F), lambda i: (0, 0))
_w_spec = pl.BlockSpec((F, F), lambda i: (0, 0))
_row_shape = jax.ShapeDtypeStruct((N, F), jnp.float32)

_comb_call = pl.pallas_call(
    _comb_body,
    grid=(N // R,),
    in_specs=[_acc_spec, _row_spec, _row_spec, _b_spec, _w_spec],
    out_specs=_row_spec,
    out_shape=_row_shape,
)

_comb2_call = pl.pallas_call(
    _comb2_body,
    grid=(N // R,),
    in_specs=[_acc_spec, _row_spec, _row_spec, _b_spec, _w_spec, _w_spec],
    out_specs=(_row_spec, _row_spec),
    out_shape=(_row_shape, _row_shape),
)

_final_call = pl.pallas_call(
    _final_body,
    grid=(N // R,),
    in_specs=[_acc_spec, _row_spec, _row_spec, _b_spec],
    out_specs=_row_spec,
    out_shape=_row_shape,
)


def _gram_body(a_ref, b_ref, o_ref):
    o_ref[...] = lax.dot_general(
        a_ref[...], b_ref[...], (((1,), (1,)), ((), ())),
        preferred_element_type=jnp.float32, precision=_HIGH)


_gram_call = pl.pallas_call(
    _gram_body,
    grid=(N // BI,),
    in_specs=[
        pl.BlockSpec((BI, F), lambda i: (i, 0)),
        pl.BlockSpec((N, F), lambda i: (0, 0)),
    ],
    out_specs=pl.BlockSpec((BI, N), lambda i: (i, 0)),
    out_shape=jax.ShapeDtypeStruct((N, N), jnp.float32),
    compiler_params=pltpu.CompilerParams(
        dimension_semantics=("arbitrary",)),
)


def kernel(x, edge_index, W1e, b1e, W2e, b2e, Wa1, ba1, Wa2, ba2, Ws1, bs1):
    src = edge_index[0].astype(jnp.int32)
    dst = edge_index[1].astype(jnp.int32)
    pad = EP - E
    srcb = jnp.concatenate([src, jnp.zeros((pad,), jnp.int32)])
    dstb = jnp.concatenate([dst, jnp.full((pad,), N, jnp.int32)])
    srcb = srcb.reshape(NW, CPT, CH)
    dstb = dstb.reshape(NW, CPT, CH)

    def conv(t):
        return _sc_scatter(t, srcb, dstb)

    degp = _sc_deg(dstb)
    dinv = _dinv_call(degp)
    t1 = _prep_call(x, W1e, dinv)
    a1 = conv(t1)
    t2 = _comb_call(a1, t1, dinv, b1e.reshape(1, F), W2e)
    a2 = conv(t2)
    t3, t5 = _comb2_call(a2, t2, dinv, b2e.reshape(1, F), Wa1, Ws1)
    a5 = conv(t5)
    s = _final_call(a5, t5, dinv, bs1.reshape(1, F))
    a3 = conv(t3)
    A_hat = _gram_call(s, s)
    t4 = _comb_call(a3, t3, dinv, ba1.reshape(1, F), Wa2)
    a4 = conv(t4)
    X_hat = _final_call(a4, t4, dinv, ba2.reshape(1, F))
    return (A_hat, X_hat)


# speedup vs baseline: 2.6279x; 2.6279x over previous
import jax
import jax.numpy as jnp
from jax import lax
from jax.experimental import pallas as pl
from jax.experimental.pallas import tpu as pltpu
from jax.experimental.pallas import tpu_sc as plsc

N = 10000
F = 128
E = 320000
NC = 2
NS = 16
NW = NC * NS
CH = 128
CPT = -(-E // (CH * NW))
EP = NW * CPT * CH
T = 10240
RPS = T // NS

_HIGH = lax.Precision.HIGHEST

_sc_mesh = plsc.VectorSubcoreMesh(
    core_axis_name="c", subcore_axis_name="s", num_cores=NC, num_subcores=NS
)


ZR = 16


def _zero_slice(zb, acc, s):
    zv = jnp.zeros((16,), jnp.float32)
    for i in range(ZR):
        for j in range(F // 16):
            zb[i, pl.ds(j * 16, 16)] = zv

    for r in range(RPS // ZR):
        pltpu.sync_copy(zb, acc.at[pl.ds(s * RPS + r * ZR, ZR)])


def _sc_scatter_body(tp, srcb, dstb, out, sidx, didx, rows, zb, acc, sem):
    c = lax.axis_index("c")
    s = lax.axis_index("s")
    wid = s * NC + c
    pltpu.sync_copy(srcb.at[wid], sidx)
    pltpu.sync_copy(dstb.at[wid], didx)
    _zero_slice(zb, acc, s)
    plsc.subcore_barrier()

    def step(j, carry):
        pltpu.async_copy(tp.at[sidx.at[j]], rows, sem).wait()
        pltpu.sync_copy(rows, acc.at[didx.at[j]], add=True)
        return carry

    lax.fori_loop(0, CPT, step, 0)
    plsc.subcore_barrier()
    pltpu.sync_copy(acc.at[pl.ds(s * RPS, RPS)], out.at[c, pl.ds(s * RPS, RPS)])


_sc_scatter = pl.kernel(
    _sc_scatter_body,
    out_type=jax.ShapeDtypeStruct((NC, T, F), jnp.float32),
    mesh=_sc_mesh,
    scratch_types=[
        pltpu.VMEM((CPT, CH), jnp.int32),
        pltpu.VMEM((CPT, CH), jnp.int32),
        pltpu.VMEM((CH, F), jnp.float32),
        pltpu.VMEM((ZR, F), jnp.float32),
        pltpu.VMEM_SHARED((T, F), jnp.float32),
        pltpu.SemaphoreType.DMA,
    ],
)


def _sc_deg_body(dstb, out, didx, ones, zb, deg):
    c = lax.axis_index("c")
    s = lax.axis_index("s")
    wid = s * NC + c
    pltpu.sync_copy(dstb.at[wid], didx)
    ov = jnp.full((16,), 1.0, jnp.float32)

    for i in range(CH):
        for j in range(F // 16):
            ones[i, pl.ds(j * 16, 16)] = ov
    _zero_slice(zb, deg, s)
    plsc.subcore_barrier()

    def step(j, carry):
        pltpu.sync_copy(ones, deg.at[didx.at[j]], add=True)
        return carry

    lax.fori_loop(0, CPT, step, 0)
    plsc.subcore_barrier()
    pltpu.sync_copy(deg.at[pl.ds(s * RPS, RPS)], out.at[c, pl.ds(s * RPS, RPS)])


_sc_deg = pl.kernel(
    _sc_deg_body,
    out_type=jax.ShapeDtypeStruct((NC, T, F), jnp.float32),
    mesh=_sc_mesh,
    scratch_types=[
        pltpu.VMEM((CPT, CH), jnp.int32),
        pltpu.VMEM((CH, F), jnp.float32),
        pltpu.VMEM((ZR, F), jnp.float32),
        pltpu.VMEM_SHARED((T, F), jnp.float32),
    ],
)


R = 1000
BI = 200


def _dinv_body(degp_ref, out_ref):
    cnt = degp_ref[0, :, 0:1] + degp_ref[1, :, 0:1] + 1.0
    out_ref[...] = jnp.broadcast_to(lax.rsqrt(cnt), (R, F))


_dinv_call = pl.pallas_call(
    _dinv_body,
    grid=(N // R,),
    in_specs=[pl.BlockSpec((NC, R, F), lambda i: (0, i, 0))],
    out_specs=pl.BlockSpec((R, F), lambda i: (i, 0)),
    out_shape=jax.ShapeDtypeStruct((N, F), jnp.float32),
)


def _prep_body(x_ref, w_ref, dinv_ref, out_ref):
    out_ref[...] = dinv_ref[...] * jnp.dot(
        x_ref[...], w_ref[...], preferred_element_type=jnp.float32,
        precision=_HIGH)


_prep_call = pl.pallas_call(
    _prep_body,
    grid=(N // R,),
    in_specs=[
        pl.BlockSpec((R, F), lambda i: (i, 0)),
        pl.BlockSpec((F, F), lambda i: (0, 0)),
        pl.BlockSpec((R, F), lambda i: (i, 0)),
    ],
    out_specs=pl.BlockSpec((R, F), lambda i: (i, 0)),
    out_shape=jax.ShapeDtypeStruct((N, F), jnp.float32),
)


def _relu_combine(acc_ref, tp_ref, dinv_ref, b_ref):
    dv = dinv_ref[...]
    return dv, jnp.maximum(
        dv * (acc_ref[0] + acc_ref[1] + tp_ref[...]) + b_ref[...], 0.0)


def _comb_body(acc_ref, tp_ref, dinv_ref, b_ref, w_ref, out_ref):
    dv, h = _relu_combine(acc_ref, tp_ref, dinv_ref, b_ref)
    out_ref[...] = dv * jnp.dot(
        h, w_ref[...], preferred_element_type=jnp.float32, precision=_HIGH)


def _comb2_body(acc_ref, tp_ref, dinv_ref, b_ref, w1_ref, w2_ref,
                o1_ref, o2_ref):
    dv, h = _relu_combine(acc_ref, tp_ref, dinv_ref, b_ref)
    o1_ref[...] = dv * jnp.dot(
        h, w1_ref[...], preferred_element_type=jnp.float32, precision=_HIGH)
    o2_ref[...] = dv * jnp.dot(
        h, w2_ref[...], preferred_element_type=jnp.float32, precision=_HIGH)


def _final_body(acc_ref, tp_ref, dinv_ref, b_ref, out_ref):
    _, h = _relu_combine(acc_ref, tp_ref, dinv_ref, b_ref)
    out_ref[...] = h


_acc_spec = pl.BlockSpec((NC, R, F), lambda i: (0, i, 0))
_row_spec = pl.BlockSpec((R, F), lambda i: (i, 0))
_b_spec = pl.BlockSpec((1, F), lambda i: (0, 0))
_w_spec = pl.BlockSpec((F, F), lambda i: (0, 0))
_row_shape = jax.ShapeDtypeStruct((N, F), jnp.float32)

_comb_call = pl.pallas_call(
    _comb_body,
    grid=(N // R,),
    in_specs=[_acc_spec, _row_spec, _row_spec, _b_spec, _w_spec],
    out_specs=_row_spec,
    out_shape=_row_shape,
)

_comb2_call = pl.pallas_call(
    _comb2_body,
    grid=(N // R,),
    in_specs=[_acc_spec, _row_spec, _row_spec, _b_spec, _w_spec, _w_spec],
    out_specs=(_row_spec, _row_spec),
    out_shape=(_row_shape, _row_shape),
)

_final_call = pl.pallas_call(
    _final_body,
    grid=(N // R,),
    in_specs=[_acc_spec, _row_spec, _row_spec, _b_spec],
    out_specs=_row_spec,
    out_shape=_row_shape,
)


def _gram_body(a_ref, b_ref, o_ref):
    o_ref[...] = lax.dot_general(
        a_ref[...], b_ref[...], (((1,), (1,)), ((), ())),
        preferred_element_type=jnp.float32, precision=_HIGH)


_gram_call = pl.pallas_call(
    _gram_body,
    grid=(N // BI,),
    in_specs=[
        pl.BlockSpec((BI, F), lambda i: (i, 0)),
        pl.BlockSpec((N, F), lambda i: (0, 0)),
    ],
    out_specs=pl.BlockSpec((BI, N), lambda i: (i, 0)),
    out_shape=jax.ShapeDtypeStruct((N, N), jnp.float32),
    compiler_params=pltpu.CompilerParams(
        dimension_semantics=("arbitrary",)),
)


def kernel(x, edge_index, W1e, b1e, W2e, b2e, Wa1, ba1, Wa2, ba2, Ws1, bs1):
    src = edge_index[0].astype(jnp.int32)
    dst = edge_index[1].astype(jnp.int32)
    pad = EP - E
    ar = jnp.arange(pad, dtype=jnp.int32)
    srcb = jnp.concatenate([src, ar % N])
    dstb = jnp.concatenate([dst, N + ar % (T - N)])
    srcb = srcb.reshape(NW, CPT, CH)
    dstb = dstb.reshape(NW, CPT, CH)

    def conv(t):
        return _sc_scatter(t, srcb, dstb)

    degp = _sc_deg(dstb)
    dinv = _dinv_call(degp)
    t1 = _prep_call(x, W1e, dinv)
    a1 = conv(t1)
    t2 = _comb_call(a1, t1, dinv, b1e.reshape(1, F), W2e)
    a2 = conv(t2)
    t3, t5 = _comb2_call(a2, t2, dinv, b2e.reshape(1, F), Wa1, Ws1)
    a5 = conv(t5)
    s = _final_call(a5, t5, dinv, bs1.reshape(1, F))
    a3 = conv(t3)
    A_hat = _gram_call(s, s)
    t4 = _comb_call(a3, t3, dinv, ba1.reshape(1, F), Wa2)
    a4 = conv(t4)
    X_hat = _final_call(a4, t4, dinv, ba2.reshape(1, F))
    return (A_hat, X_hat)


# speedup vs baseline: 2.6553x; 1.0104x over previous
import jax
import jax.numpy as jnp
from jax import lax
from jax.experimental import pallas as pl
from jax.experimental.pallas import tpu as pltpu
from jax.experimental.pallas import tpu_sc as plsc

N = 10000
F = 128
E = 320000
NC = 2
NS = 16
NW = NC * NS
CH = 128
CPT = -(-E // (CH * NW))
EP = NW * CPT * CH
T = 10240
RPS = T // NS

_HIGH = lax.Precision.HIGHEST

_sc_mesh = plsc.VectorSubcoreMesh(
    core_axis_name="c", subcore_axis_name="s", num_cores=NC, num_subcores=NS
)


ZR = 16


def _zero_slice(zb, acc, s):
    zv = jnp.zeros((16,), jnp.float32)
    for i in range(ZR):
        for j in range(F // 16):
            zb[i, pl.ds(j * 16, 16)] = zv

    for r in range(RPS // ZR):
        pltpu.sync_copy(zb, acc.at[pl.ds(s * RPS + r * ZR, ZR)])


def _sc_scatter_body(tp, srcb, dstb, out, sidx, didx, rows, zb, acc, sem):
    c = lax.axis_index("c")
    s = lax.axis_index("s")
    wid = s * NC + c
    pltpu.sync_copy(srcb.at[wid], sidx)
    pltpu.sync_copy(dstb.at[wid], didx)
    _zero_slice(zb, acc, s)
    plsc.subcore_barrier()

    def step(j, carry):
        pltpu.async_copy(tp.at[sidx.at[j]], rows, sem).wait()
        pltpu.sync_copy(rows, acc.at[didx.at[j]], add=True)
        return carry

    lax.fori_loop(0, CPT, step, 0)
    plsc.subcore_barrier()
    pltpu.sync_copy(acc.at[pl.ds(s * RPS, RPS)], out.at[c, pl.ds(s * RPS, RPS)])


_sc_scatter = pl.kernel(
    _sc_scatter_body,
    out_type=jax.ShapeDtypeStruct((NC, T, F), jnp.float32),
    mesh=_sc_mesh,
    scratch_types=[
        pltpu.VMEM((CPT, CH), jnp.int32),
        pltpu.VMEM((CPT, CH), jnp.int32),
        pltpu.VMEM((CH, F), jnp.float32),
        pltpu.VMEM((ZR, F), jnp.float32),
        pltpu.VMEM_SHARED((T, F), jnp.float32),
        pltpu.SemaphoreType.DMA,
    ],
)


def _sc_deg_body(dstb, out, didx, ones, zb, deg):
    c = lax.axis_index("c")
    s = lax.axis_index("s")
    wid = s * NC + c
    pltpu.sync_copy(dstb.at[wid], didx)
    ov = jnp.full((16,), 1.0, jnp.float32)

    for i in range(CH):
        for j in range(F // 16):
            ones[i, pl.ds(j * 16, 16)] = ov
    _zero_slice(zb, deg, s)
    plsc.subcore_barrier()

    def step(j, carry):
        pltpu.sync_copy(ones, deg.at[didx.at[j]], add=True)
        return carry

    lax.fori_loop(0, CPT, step, 0)
    plsc.subcore_barrier()
    pltpu.sync_copy(deg.at[pl.ds(s * RPS, RPS)], out.at[c, pl.ds(s * RPS, RPS)])


_sc_deg = pl.kernel(
    _sc_deg_body,
    out_type=jax.ShapeDtypeStruct((NC, T, F), jnp.float32),
    mesh=_sc_mesh,
    scratch_types=[
        pltpu.VMEM((CPT, CH), jnp.int32),
        pltpu.VMEM((CH, F), jnp.float32),
        pltpu.VMEM((ZR, F), jnp.float32),
        pltpu.VMEM_SHARED((T, F), jnp.float32),
    ],
)


R = 1000
BI = 200


def _mm_body(x_ref, w_ref, out_ref):
    out_ref[...] = jnp.dot(
        x_ref[...], w_ref[...], preferred_element_type=jnp.float32,
        precision=_HIGH)


_mm_call = pl.pallas_call(
    _mm_body,
    grid=(N // R,),
    in_specs=[
        pl.BlockSpec((R, F), lambda i: (i, 0)),
        pl.BlockSpec((F, F), lambda i: (0, 0)),
    ],
    out_specs=pl.BlockSpec((R, F), lambda i: (i, 0)),
    out_shape=jax.ShapeDtypeStruct((N, F), jnp.float32),
)


def _dinv_body(degp_ref, xw_ref, dinv_ref, t1_ref):
    cnt = degp_ref[0, :, 0:1] + degp_ref[1, :, 0:1] + 1.0
    dv = jnp.broadcast_to(lax.rsqrt(cnt), (R, F))
    dinv_ref[...] = dv
    t1_ref[...] = dv * xw_ref[...]


_dinv_call = pl.pallas_call(
    _dinv_body,
    grid=(N // R,),
    in_specs=[
        pl.BlockSpec((NC, R, F), lambda i: (0, i, 0)),
        pl.BlockSpec((R, F), lambda i: (i, 0)),
    ],
    out_specs=(pl.BlockSpec((R, F), lambda i: (i, 0)),
               pl.BlockSpec((R, F), lambda i: (i, 0))),
    out_shape=(jax.ShapeDtypeStruct((N, F), jnp.float32),
               jax.ShapeDtypeStruct((N, F), jnp.float32)),
)


def _relu_combine(acc_ref, tp_ref, dinv_ref, b_ref):
    dv = dinv_ref[...]
    return dv, jnp.maximum(
        dv * (acc_ref[0] + acc_ref[1] + tp_ref[...]) + b_ref[...], 0.0)


def _comb_body(acc_ref, tp_ref, dinv_ref, b_ref, w_ref, out_ref):
    dv, h = _relu_combine(acc_ref, tp_ref, dinv_ref, b_ref)
    out_ref[...] = dv * jnp.dot(
        h, w_ref[...], preferred_element_type=jnp.float32, precision=_HIGH)


def _comb2_body(acc_ref, tp_ref, dinv_ref, b_ref, w1_ref, w2_ref,
                o1_ref, o2_ref):
    dv, h = _relu_combine(acc_ref, tp_ref, dinv_ref, b_ref)
    o1_ref[...] = dv * jnp.dot(
        h, w1_ref[...], preferred_element_type=jnp.float32, precision=_HIGH)
    o2_ref[...] = dv * jnp.dot(
        h, w2_ref[...], preferred_element_type=jnp.float32, precision=_HIGH)


def _final_body(acc_ref, tp_ref, dinv_ref, b_ref, out_ref):
    _, h = _relu_combine(acc_ref, tp_ref, dinv_ref, b_ref)
    out_ref[...] = h


_acc_spec = pl.BlockSpec((NC, R, F), lambda i: (0, i, 0))
_row_spec = pl.BlockSpec((R, F), lambda i: (i, 0))
_b_spec = pl.BlockSpec((1, F), lambda i: (0, 0))
_w_spec = pl.BlockSpec((F, F), lambda i: (0, 0))
_row_shape = jax.ShapeDtypeStruct((N, F), jnp.float32)

_comb_call = pl.pallas_call(
    _comb_body,
    grid=(N // R,),
    in_specs=[_acc_spec, _row_spec, _row_spec, _b_spec, _w_spec],
    out_specs=_row_spec,
    out_shape=_row_shape,
)

_comb2_call = pl.pallas_call(
    _comb2_body,
    grid=(N // R,),
    in_specs=[_acc_spec, _row_spec, _row_spec, _b_spec, _w_spec, _w_spec],
    out_specs=(_row_spec, _row_spec),
    out_shape=(_row_shape, _row_shape),
)

_final_call = pl.pallas_call(
    _final_body,
    grid=(N // R,),
    in_specs=[_acc_spec, _row_spec, _row_spec, _b_spec],
    out_specs=_row_spec,
    out_shape=_row_shape,
)


def _gram_body(a_ref, b_ref, o_ref):
    o_ref[...] = lax.dot_general(
        a_ref[...], b_ref[...], (((1,), (1,)), ((), ())),
        preferred_element_type=jnp.float32, precision=_HIGH)


_gram_call = pl.pallas_call(
    _gram_body,
    grid=(N // BI,),
    in_specs=[
        pl.BlockSpec((BI, F), lambda i: (i, 0)),
        pl.BlockSpec((N, F), lambda i: (0, 0)),
    ],
    out_specs=pl.BlockSpec((BI, N), lambda i: (i, 0)),
    out_shape=jax.ShapeDtypeStruct((N, N), jnp.float32),
    compiler_params=pltpu.CompilerParams(
        dimension_semantics=("arbitrary",)),
)


def kernel(x, edge_index, W1e, b1e, W2e, b2e, Wa1, ba1, Wa2, ba2, Ws1, bs1):
    src = edge_index[0].astype(jnp.int32)
    dst = edge_index[1].astype(jnp.int32)
    pad = EP - E
    ar = jnp.arange(pad, dtype=jnp.int32)
    srcb = jnp.concatenate([src, ar % N])
    dstb = jnp.concatenate([dst, N + ar % (T - N)])
    srcb = srcb.reshape(NW, CPT, CH)
    dstb = dstb.reshape(NW, CPT, CH)

    def conv(t):
        return _sc_scatter(t, srcb, dstb)

    xw = _mm_call(x, W1e)
    degp = _sc_deg(dstb)
    dinv, t1 = _dinv_call(degp, xw)
    a1 = conv(t1)
    t2 = _comb_call(a1, t1, dinv, b1e.reshape(1, F), W2e)
    a2 = conv(t2)
    t3, t5 = _comb2_call(a2, t2, dinv, b2e.reshape(1, F), Wa1, Ws1)
    a5 = conv(t5)
    s = _final_call(a5, t5, dinv, bs1.reshape(1, F))
    a3 = conv(t3)
    A_hat = _gram_call(s, s)
    t4 = _comb_call(a3, t3, dinv, ba1.reshape(1, F), Wa2)
    a4 = conv(t4)
    X_hat = _final_call(a4, t4, dinv, ba2.reshape(1, F))
    return (A_hat, X_hat)


# speedup vs baseline: 2.9644x; 1.1164x over previous
import jax
import jax.numpy as jnp
from jax import lax
from jax.experimental import pallas as pl
from jax.experimental.pallas import tpu as pltpu
from jax.experimental.pallas import tpu_sc as plsc

N = 10000
F = 128
E = 320000
NC = 2
NS = 16
NW = NC * NS
CH = 128
CPT = -(-E // (CH * NW))
EP = NW * CPT * CH
T = 10240
RPS = T // NS

_HIGH = lax.Precision.HIGHEST

_sc_mesh = plsc.VectorSubcoreMesh(
    core_axis_name="c", subcore_axis_name="s", num_cores=NC, num_subcores=NS
)


ZR = 16


def _zero_slice(zb, acc, s):
    zv = jnp.zeros((16,), jnp.float32)
    for i in range(ZR):
        for j in range(F // 16):
            zb[i, pl.ds(j * 16, 16)] = zv

    for r in range(RPS // ZR):
        pltpu.sync_copy(zb, acc.at[pl.ds(s * RPS + r * ZR, ZR)])


def _sc_scatter_body(tp, srcb, dstb, out, sidx, didx, rows, zb, acc, sem):
    c = lax.axis_index("c")
    s = lax.axis_index("s")
    wid = s * NC + c
    pltpu.sync_copy(srcb.at[wid], sidx)
    pltpu.sync_copy(dstb.at[wid], didx)
    _zero_slice(zb, acc, s)
    plsc.subcore_barrier()

    def step(j, carry):
        pltpu.async_copy(tp.at[sidx.at[j]], rows, sem).wait()
        pltpu.sync_copy(rows, acc.at[didx.at[j]], add=True)
        return carry

    lax.fori_loop(0, CPT, step, 0)
    plsc.subcore_barrier()
    pltpu.sync_copy(acc.at[pl.ds(s * RPS, RPS)], out.at[c, pl.ds(s * RPS, RPS)])


_sc_scatter = pl.kernel(
    _sc_scatter_body,
    out_type=jax.ShapeDtypeStruct((NC, T, F), jnp.float32),
    mesh=_sc_mesh,
    scratch_types=[
        pltpu.VMEM((CPT, CH), jnp.int32),
        pltpu.VMEM((CPT, CH), jnp.int32),
        pltpu.VMEM((CH, F), jnp.float32),
        pltpu.VMEM((ZR, F), jnp.float32),
        pltpu.VMEM_SHARED((T, F), jnp.float32),
        pltpu.SemaphoreType.DMA,
    ],
)


def _sc_deg_body(dstb, out, didx, ones, zb, deg):
    c = lax.axis_index("c")
    s = lax.axis_index("s")
    wid = s * NC + c
    pltpu.sync_copy(dstb.at[wid], didx)
    ov = jnp.full((16,), 1.0, jnp.float32)

    for i in range(CH):
        for j in range(F // 16):
            ones[i, pl.ds(j * 16, 16)] = ov
    _zero_slice(zb, deg, s)
    plsc.subcore_barrier()

    def step(j, carry):
        pltpu.sync_copy(ones, deg.at[didx.at[j]], add=True)
        return carry

    lax.fori_loop(0, CPT, step, 0)
    plsc.subcore_barrier()
    pltpu.sync_copy(deg.at[pl.ds(s * RPS, RPS)], out.at[c, pl.ds(s * RPS, RPS)])


_sc_deg = pl.kernel(
    _sc_deg_body,
    out_type=jax.ShapeDtypeStruct((NC, T, F), jnp.float32),
    mesh=_sc_mesh,
    scratch_types=[
        pltpu.VMEM((CPT, CH), jnp.int32),
        pltpu.VMEM((CH, F), jnp.float32),
        pltpu.VMEM((ZR, F), jnp.float32),
        pltpu.VMEM_SHARED((T, F), jnp.float32),
    ],
)


R = 1000
BI = 200


def _mm_body(x_ref, w_ref, out_ref):
    out_ref[...] = jnp.dot(
        x_ref[...], w_ref[...], preferred_element_type=jnp.float32,
        precision=_HIGH)


_mm_call = pl.pallas_call(
    _mm_body,
    grid=(N // R,),
    in_specs=[
        pl.BlockSpec((R, F), lambda i: (i, 0)),
        pl.BlockSpec((F, F), lambda i: (0, 0)),
    ],
    out_specs=pl.BlockSpec((R, F), lambda i: (i, 0)),
    out_shape=jax.ShapeDtypeStruct((N, F), jnp.float32),
)


def _dinv_body(degp_ref, xw_ref, dinv_ref, t1_ref):
    cnt = degp_ref[0, :, 0:1] + degp_ref[1, :, 0:1] + 1.0
    dv = jnp.broadcast_to(lax.rsqrt(cnt), (R, F))
    dinv_ref[...] = dv
    t1_ref[...] = dv * xw_ref[...]


_dinv_call = pl.pallas_call(
    _dinv_body,
    grid=(N // R,),
    in_specs=[
        pl.BlockSpec((NC, R, F), lambda i: (0, i, 0)),
        pl.BlockSpec((R, F), lambda i: (i, 0)),
    ],
    out_specs=(pl.BlockSpec((R, F), lambda i: (i, 0)),
               pl.BlockSpec((R, F), lambda i: (i, 0))),
    out_shape=(jax.ShapeDtypeStruct((N, F), jnp.float32),
               jax.ShapeDtypeStruct((N, F), jnp.float32)),
)


def _relu_combine(acc_ref, tp_ref, dinv_ref, b_ref):
    dv = dinv_ref[...]
    return dv, jnp.maximum(
        dv * (acc_ref[0] + acc_ref[1] + tp_ref[...]) + b_ref[...], 0.0)


def _comb_body(acc_ref, tp_ref, dinv_ref, b_ref, w_ref, out_ref):
    dv, h = _relu_combine(acc_ref, tp_ref, dinv_ref, b_ref)
    out_ref[...] = dv * jnp.dot(
        h, w_ref[...], preferred_element_type=jnp.float32, precision=_HIGH)


def _comb2_body(acc_ref, tp_ref, dinv_ref, b_ref, w1_ref, w2_ref,
                o1_ref, o2_ref):
    dv, h = _relu_combine(acc_ref, tp_ref, dinv_ref, b_ref)
    o1_ref[...] = dv * jnp.dot(
        h, w1_ref[...], preferred_element_type=jnp.float32, precision=_HIGH)
    o2_ref[...] = dv * jnp.dot(
        h, w2_ref[...], preferred_element_type=jnp.float32, precision=_HIGH)


def _final_body(acc_ref, tp_ref, dinv_ref, b_ref, out_ref):
    _, h = _relu_combine(acc_ref, tp_ref, dinv_ref, b_ref)
    out_ref[...] = h


_acc_spec = pl.BlockSpec((NC, R, F), lambda i: (0, i, 0))
_row_spec = pl.BlockSpec((R, F), lambda i: (i, 0))
_b_spec = pl.BlockSpec((1, F), lambda i: (0, 0))
_w_spec = pl.BlockSpec((F, F), lambda i: (0, 0))
_row_shape = jax.ShapeDtypeStruct((N, F), jnp.float32)

_comb_call = pl.pallas_call(
    _comb_body,
    grid=(N // R,),
    in_specs=[_acc_spec, _row_spec, _row_spec, _b_spec, _w_spec],
    out_specs=_row_spec,
    out_shape=_row_shape,
)

_comb2_call = pl.pallas_call(
    _comb2_body,
    grid=(N // R,),
    in_specs=[_acc_spec, _row_spec, _row_spec, _b_spec, _w_spec, _w_spec],
    out_specs=(_row_spec, _row_spec),
    out_shape=(_row_shape, _row_shape),
)

_final_call = pl.pallas_call(
    _final_body,
    grid=(N // R,),
    in_specs=[_acc_spec, _row_spec, _row_spec, _b_spec],
    out_specs=_row_spec,
    out_shape=_row_shape,
)


def _gram_body(a_ref, b_ref, o_ref):
    o_ref[...] = lax.dot_general(
        a_ref[...], b_ref[...], (((1,), (1,)), ((), ())),
        preferred_element_type=jnp.float32)


_gram_call = pl.pallas_call(
    _gram_body,
    grid=(N // BI,),
    in_specs=[
        pl.BlockSpec((BI, F), lambda i: (i, 0)),
        pl.BlockSpec((N, F), lambda i: (0, 0)),
    ],
    out_specs=pl.BlockSpec((BI, N), lambda i: (i, 0)),
    out_shape=jax.ShapeDtypeStruct((N, N), jnp.float32),
    compiler_params=pltpu.CompilerParams(
        dimension_semantics=("arbitrary",)),
)


def kernel(x, edge_index, W1e, b1e, W2e, b2e, Wa1, ba1, Wa2, ba2, Ws1, bs1):
    src = edge_index[0].astype(jnp.int32)
    dst = edge_index[1].astype(jnp.int32)
    pad = EP - E
    ar = jnp.arange(pad, dtype=jnp.int32)
    srcb = jnp.concatenate([src, ar % N])
    dstb = jnp.concatenate([dst, N + ar % (T - N)])
    srcb = srcb.reshape(NW, CPT, CH)
    dstb = dstb.reshape(NW, CPT, CH)

    def conv(t):
        return _sc_scatter(t, srcb, dstb)

    xw = _mm_call(x, W1e)
    degp = _sc_deg(dstb)
    dinv, t1 = _dinv_call(degp, xw)
    a1 = conv(t1)
    t2 = _comb_call(a1, t1, dinv, b1e.reshape(1, F), W2e)
    a2 = conv(t2)
    t3, t5 = _comb2_call(a2, t2, dinv, b2e.reshape(1, F), Wa1, Ws1)
    a5 = conv(t5)
    s = _final_call(a5, t5, dinv, bs1.reshape(1, F))
    A_hat = _gram_call(s, s)
    a3 = conv(t3)
    t4 = _comb_call(a3, t3, dinv, ba1.reshape(1, F), Wa2)
    a4 = conv(t4)
    X_hat = _final_call(a4, t4, dinv, ba2.reshape(1, F))
    return (A_hat, X_hat)
